# baseline (speedup 1.0000x reference)
import functools

import jax
import jax.numpy as jnp
import numpy as np
from jax import lax
from jax.experimental import pallas as pl
from jax.experimental.pallas import tpu as pltpu
from jax.experimental.pallas import tpu_sc as plsc

N = 10000
E = 320000
NG = 64
D = 128

NW = 32
LANE = 128
RPT = 80
EPT = RPT * LANE
EPAD = NW * EPT
NPAD = 10240
SLC = NPAD // 16


def _proj_body(x_ref, w_ref, asr_ref, adr_ref, cw_ref, o_ref):
    h = jnp.dot(x_ref[...], w_ref[...], preferred_element_type=jnp.float32)
    a = jnp.concatenate([asr_ref[...], adr_ref[...], cw_ref[...],
                         jnp.zeros((5, D), jnp.float32)], axis=0)
    o_ref[...] = lax.dot_general(
        h, a, (((1,), (1,)), ((), ())), preferred_element_type=jnp.float32)


def _edge_body(src_hbm, dst_hbm, asp_hbm, adp_hbm, vp_hbm, s_out, n_out,
               asp_v, adp_v, vp_v, src_v, dst_v, p_v, q_v, z_v,
               acc_s, acc_n):
    c = lax.axis_index("c")
    sid = lax.axis_index("s")
    wid = c * 16 + sid
    base = wid * RPT
    pltpu.sync_copy(src_hbm.at[pl.ds(base, RPT)], src_v)
    pltpu.sync_copy(dst_hbm.at[pl.ds(base, RPT)], dst_v)
    pltpu.sync_copy(asp_hbm, asp_v)
    pltpu.sync_copy(adp_hbm, adp_v)
    pltpu.sync_copy(vp_hbm, vp_v)

    zero16 = jnp.zeros((16,), jnp.float32)

    def zb(i, carry):
        z_v[pl.ds(i * 16, 16)] = zero16
        return carry

    lax.fori_loop(0, SLC // 16, zb, 0)
    pltpu.sync_copy(z_v, acc_s.at[pl.ds(sid * SLC, SLC)])
    pltpu.sync_copy(z_v, acc_n.at[pl.ds(sid * SLC, SLC)])
    plsc.subcore_barrier()

    def row(j, carry):
        for k in range(LANE // 16):
            sl = pl.ds(k * 16, 16)
            si = src_v[j, sl]
            di = dst_v[j, sl]
            a_s = plsc.load_gather(asp_v, [si])
            a_d = plsc.load_gather(adp_v, [di])
            vv = plsc.load_gather(vp_v, [si])
            z = a_s + a_d
            p = jnp.exp(jnp.maximum(z, z * 0.2))
            p_v[j, sl] = p
            q_v[j, sl] = p * vv
        return carry

    lax.fori_loop(0, RPT, row, 0)

    def srow(j, carry):
        pltpu.sync_copy(p_v.at[j], acc_s.at[dst_v.at[j]], add=True)
        pltpu.sync_copy(q_v.at[j], acc_n.at[dst_v.at[j]], add=True)
        return carry

    lax.fori_loop(0, RPT, srow, 0)
    plsc.subcore_barrier()

    pltpu.sync_copy(acc_s.at[pl.ds(sid * SLC, SLC)],
                    s_out.at[c, pl.ds(sid * SLC, SLC)])
    pltpu.sync_copy(acc_n.at[pl.ds(sid * SLC, SLC)],
                    n_out.at[c, pl.ds(sid * SLC, SLC)])


_edge_call = functools.partial(
    pl.kernel,
    out_type=(jax.ShapeDtypeStruct((2, NPAD), jnp.float32),
              jax.ShapeDtypeStruct((2, NPAD), jnp.float32)),
    mesh=plsc.VectorSubcoreMesh(core_axis_name="c", subcore_axis_name="s"),
    compiler_params=pltpu.CompilerParams(needs_layout_passes=False),
    scratch_types=[
        pltpu.VMEM((NPAD,), jnp.float32),
        pltpu.VMEM((NPAD,), jnp.float32),
        pltpu.VMEM((NPAD,), jnp.float32),
        pltpu.VMEM((RPT, LANE), jnp.int32),
        pltpu.VMEM((RPT, LANE), jnp.int32),
        pltpu.VMEM((RPT, LANE), jnp.float32),
        pltpu.VMEM((RPT, LANE), jnp.float32),
        pltpu.VMEM((SLC,), jnp.float32),
        pltpu.VMEM_SHARED((NPAD,), jnp.float32),
        pltpu.VMEM_SHARED((NPAD,), jnp.float32),
    ],
)


def _final_body(s0_ref, s1_ref, n0_ref, n1_ref, asp_ref, adp_ref, vp_ref,
                b_ref, bias_ref, clsw_ref, clsb_ref, o_ref):
    z = asp_ref[...] + adp_ref[...]
    sp = jnp.exp(jnp.maximum(z, z * 0.2))
    stot = s0_ref[...] + s1_ref[...] + sp
    ntot = n0_ref[...] + n1_ref[...] + sp * vp_ref[...]
    c0 = jnp.sum(bias_ref[...] * clsw_ref[...])
    nv = ntot / (stot + 1e-16) + c0
    batch = b_ref[...]
    nv = jnp.where(batch < NG, nv, 0.0)
    gids = lax.broadcasted_iota(jnp.int32, (NG, NPAD // D, D), 0)
    eq = batch[None, :, :] == gids
    sums = jnp.sum(jnp.where(eq, nv[None, :, :], 0.0), axis=2).sum(axis=1)
    cnt = jnp.sum(eq.astype(jnp.float32), axis=2).sum(axis=1)
    logits = sums / jnp.maximum(cnt, 1.0) + clsb_ref[0, 0]
    o_ref[...] = logits[:, None]


def kernel(x, edge_index, batch, W, a_src, a_dst, bias, cls_W, cls_b):
    f32 = jnp.float32
    RB = 1000
    P = pl.pallas_call(
        _proj_body,
        grid=(N // RB,),
        in_specs=[pl.BlockSpec((RB, D), lambda i: (i, 0)),
                  pl.BlockSpec((D, D), lambda i: (0, 0)),
                  pl.BlockSpec((1, D), lambda i: (0, 0)),
                  pl.BlockSpec((1, D), lambda i: (0, 0)),
                  pl.BlockSpec((1, D), lambda i: (0, 0))],
        out_specs=pl.BlockSpec((RB, 8), lambda i: (i, 0)),
        out_shape=jax.ShapeDtypeStruct((N, 8), f32),
    )(x.astype(f32), W.astype(f32), a_src.reshape(1, D).astype(f32),
      a_dst.reshape(1, D).astype(f32), cls_W.reshape(1, D).astype(f32))
    asp = jnp.pad(P[:, 0], (0, NPAD - N))
    adp = jnp.pad(P[:, 1], (0, NPAD - N))
    vp = jnp.pad(P[:, 2], (0, NPAD - N))

    src = edge_index[0].astype(jnp.int32)
    dst = edge_index[1].astype(jnp.int32)
    npd = EPAD - E
    pad_src = np.zeros((npd,), np.int32)
    pad_dst = (N + np.arange(npd, dtype=np.int32) % 112).astype(np.int32)
    src_p = jnp.concatenate([src, jnp.asarray(pad_src)])
    dst_p = jnp.concatenate([dst, jnp.asarray(pad_dst)])
    src_p = src_p.reshape(NW * RPT, LANE)
    dst_p = dst_p.reshape(NW * RPT, LANE)

    s_part, n_part = _edge_call(_edge_body)(src_p, dst_p, asp, adp, vp)

    bpad = jnp.full((NPAD - N,), 1 << 20, jnp.int32)
    b2 = jnp.concatenate([batch.astype(jnp.int32), bpad]).reshape(NPAD // D, D)
    logits = pl.pallas_call(
        _final_body,
        out_shape=jax.ShapeDtypeStruct((NG, 1), f32),
    )(s_part[0].reshape(NPAD // D, D), s_part[1].reshape(NPAD // D, D),
      n_part[0].reshape(NPAD // D, D), n_part[1].reshape(NPAD // D, D),
      asp.reshape(NPAD // D, D), adp.reshape(NPAD // D, D),
      vp.reshape(NPAD // D, D), b2,
      bias.reshape(1, D), cls_W.reshape(1, D).astype(f32),
      cls_b.reshape(1, 1).astype(f32))
    return logits


# speedup vs baseline: 1.1722x; 1.1722x over previous
import functools

import jax
import jax.numpy as jnp
import numpy as np
from jax import lax
from jax.experimental import pallas as pl
from jax.experimental.pallas import tpu as pltpu
from jax.experimental.pallas import tpu_sc as plsc

N = 10000
E = 320000
NG = 64
D = 128

NW = 32
LANE = 128
RPT = 80
EPT = RPT * LANE
EPAD = NW * EPT
NPAD = 10240
SLC = NPAD // 16


def _proj_body(x_ref, w_ref, asr_ref, adr_ref, cw_ref, o_ref):
    h = jnp.dot(x_ref[...], w_ref[...], preferred_element_type=jnp.float32)
    a = jnp.concatenate([asr_ref[...], adr_ref[...], cw_ref[...],
                         jnp.zeros((5, D), jnp.float32)], axis=0)
    o_ref[...] = lax.dot_general(
        a, h, (((1,), (1,)), ((), ())), preferred_element_type=jnp.float32)


def _edge_body(src_hbm, dst_hbm, asp_hbm, adp_hbm, vp_hbm, s_out, n_out,
               asp_v, adp_v, vp_v, src_v, dst_v, p_v, q_v, z_v,
               acc_s, acc_n):
    c = lax.axis_index("c")
    sid = lax.axis_index("s")
    wid = c * 16 + sid
    base = wid * RPT
    pltpu.sync_copy(src_hbm.at[pl.ds(base, RPT)], src_v)
    pltpu.sync_copy(dst_hbm.at[pl.ds(base, RPT)], dst_v)
    pltpu.sync_copy(asp_hbm, asp_v)
    pltpu.sync_copy(adp_hbm, adp_v)
    pltpu.sync_copy(vp_hbm, vp_v)

    zero16 = jnp.zeros((16,), jnp.float32)

    def zb(i, carry):
        z_v[pl.ds(i * 16, 16)] = zero16
        return carry

    lax.fori_loop(0, SLC // 16, zb, 0)
    pltpu.sync_copy(z_v, acc_s.at[pl.ds(sid * SLC, SLC)])
    pltpu.sync_copy(z_v, acc_n.at[pl.ds(sid * SLC, SLC)])
    plsc.subcore_barrier()

    def row(j, carry):
        for k in range(LANE // 16):
            sl = pl.ds(k * 16, 16)
            si = src_v[j, sl]
            di = dst_v[j, sl]
            a_s = plsc.load_gather(asp_v, [si])
            a_d = plsc.load_gather(adp_v, [di])
            vv = plsc.load_gather(vp_v, [si])
            z = a_s + a_d
            p = jnp.exp(jnp.maximum(z, z * 0.2))
            p_v[j, sl] = p
            q_v[j, sl] = p * vv
        return carry

    lax.fori_loop(0, RPT, row, 0)

    def srow(j, carry):
        pltpu.sync_copy(p_v.at[j], acc_s.at[dst_v.at[j]], add=True)
        pltpu.sync_copy(q_v.at[j], acc_n.at[dst_v.at[j]], add=True)
        return carry

    lax.fori_loop(0, RPT, srow, 0)
    plsc.subcore_barrier()

    pltpu.sync_copy(acc_s.at[pl.ds(sid * SLC, SLC)],
                    s_out.at[c, pl.ds(sid * SLC, SLC)])
    pltpu.sync_copy(acc_n.at[pl.ds(sid * SLC, SLC)],
                    n_out.at[c, pl.ds(sid * SLC, SLC)])


_edge_call = functools.partial(
    pl.kernel,
    out_type=(jax.ShapeDtypeStruct((2, NPAD), jnp.float32),
              jax.ShapeDtypeStruct((2, NPAD), jnp.float32)),
    mesh=plsc.VectorSubcoreMesh(core_axis_name="c", subcore_axis_name="s"),
    compiler_params=pltpu.CompilerParams(needs_layout_passes=False),
    scratch_types=[
        pltpu.VMEM((NPAD,), jnp.float32),
        pltpu.VMEM((NPAD,), jnp.float32),
        pltpu.VMEM((NPAD,), jnp.float32),
        pltpu.VMEM((RPT, LANE), jnp.int32),
        pltpu.VMEM((RPT, LANE), jnp.int32),
        pltpu.VMEM((RPT, LANE), jnp.float32),
        pltpu.VMEM((RPT, LANE), jnp.float32),
        pltpu.VMEM((SLC,), jnp.float32),
        pltpu.VMEM_SHARED((NPAD,), jnp.float32),
        pltpu.VMEM_SHARED((NPAD,), jnp.float32),
    ],
)


def _final_body(s0_ref, s1_ref, n0_ref, n1_ref, asp_ref, adp_ref, vp_ref,
                b_ref, bias_ref, clsw_ref, clsb_ref, o_ref):
    z = asp_ref[...] + adp_ref[...]
    sp = jnp.exp(jnp.maximum(z, z * 0.2))
    stot = s0_ref[...] + s1_ref[...] + sp
    ntot = n0_ref[...] + n1_ref[...] + sp * vp_ref[...]
    c0 = jnp.sum(bias_ref[...] * clsw_ref[...])
    nv = ntot / (stot + 1e-16) + c0
    batch = b_ref[...]
    nv = jnp.where(batch < NG, nv, 0.0)
    gids = lax.broadcasted_iota(jnp.int32, (NG, NPAD), 0)
    eq = batch == gids
    sums = jnp.sum(jnp.where(eq, nv, 0.0), axis=1)
    cnt = jnp.sum(eq.astype(jnp.float32), axis=1)
    logits = sums / jnp.maximum(cnt, 1.0) + clsb_ref[0, 0]
    o_ref[...] = logits[:, None]


def kernel(x, edge_index, batch, W, a_src, a_dst, bias, cls_W, cls_b):
    f32 = jnp.float32
    P = pl.pallas_call(
        _proj_body,
        out_shape=jax.ShapeDtypeStruct((8, N), f32),
    )(x.astype(f32), W.astype(f32), a_src.reshape(1, D).astype(f32),
      a_dst.reshape(1, D).astype(f32), cls_W.reshape(1, D).astype(f32))
    asp = jnp.pad(P[0], (0, NPAD - N))
    adp = jnp.pad(P[1], (0, NPAD - N))
    vp = jnp.pad(P[2], (0, NPAD - N))

    src = edge_index[0].astype(jnp.int32)
    dst = edge_index[1].astype(jnp.int32)
    npd = EPAD - E
    pad_src = np.zeros((npd,), np.int32)
    pad_dst = (N + np.arange(npd, dtype=np.int32) % 112).astype(np.int32)
    src_p = jnp.concatenate([src, jnp.asarray(pad_src)])
    dst_p = jnp.concatenate([dst, jnp.asarray(pad_dst)])
    src_p = src_p.reshape(NW * RPT, LANE)
    dst_p = dst_p.reshape(NW * RPT, LANE)

    s_part, n_part = _edge_call(_edge_body)(src_p, dst_p, asp, adp, vp)

    b32 = batch.astype(jnp.int32)
    bpad = jnp.asarray(np.full((NPAD - N,), 1 << 20, np.int32))
    b2 = jnp.concatenate([b32, bpad]).reshape(1, NPAD)
    logits = pl.pallas_call(
        _final_body,
        out_shape=jax.ShapeDtypeStruct((NG, 1), f32),
    )(s_part[0].reshape(1, NPAD), s_part[1].reshape(1, NPAD),
      n_part[0].reshape(1, NPAD), n_part[1].reshape(1, NPAD),
      asp.reshape(1, NPAD), adp.reshape(1, NPAD),
      vp.reshape(1, NPAD), b2,
      bias.reshape(1, D), cls_W.reshape(1, D).astype(f32),
      cls_b.reshape(1, 1).astype(f32))
    return logits


# speedup vs baseline: 1.3943x; 1.1894x over previous
import functools

import jax
import jax.numpy as jnp
import numpy as np
from jax import lax
from jax.experimental import pallas as pl
from jax.experimental.pallas import tpu as pltpu
from jax.experimental.pallas import tpu_sc as plsc

N = 10000
E = 320000
NG = 64
D = 128

NW = 32
LANE = 128
RPT = 80
EPT = RPT * LANE
EPAD = NW * EPT
NPAD = 10240
SLC = NPAD // 16


def _proj_body(x_ref, w_ref, asr_ref, adr_ref, cw_ref, o_ref):
    h = jnp.dot(x_ref[...], w_ref[...], preferred_element_type=jnp.float32)
    a = jnp.concatenate([asr_ref[...], adr_ref[...], cw_ref[...],
                         jnp.zeros((5, D), jnp.float32)], axis=0)
    o_ref[...] = lax.dot_general(
        a, h, (((1,), (1,)), ((), ())), preferred_element_type=jnp.float32)


def _edge_body(e_hbm, ps_hbm, pd_hbm, asp_hbm, adp_hbm, vp_hbm, s_out, n_out,
               asp_v, adp_v, vp_v, src_v, dst_v, p_v, q_v, z_v,
               acc_s, acc_n):
    c = lax.axis_index("c")
    sid = lax.axis_index("s")
    wid = c * 16 + sid
    base = wid * RPT
    nfull = E // EPT

    @pl.when(wid < nfull)
    def _():
        pltpu.sync_copy(e_hbm.at[0, pl.ds(base, RPT)], src_v)
        pltpu.sync_copy(e_hbm.at[1, pl.ds(base, RPT)], dst_v)

    @pl.when(wid == nfull)
    def _():
        tail = E // LANE - nfull * RPT
        t8 = tail + 4
        pltpu.sync_copy(e_hbm.at[0, pl.ds(nfull * RPT, t8)],
                        src_v.at[pl.ds(0, t8)])
        pltpu.sync_copy(e_hbm.at[1, pl.ds(nfull * RPT, t8)],
                        dst_v.at[pl.ds(0, t8)])
        pltpu.sync_copy(ps_hbm.at[pl.ds(0, RPT - t8)],
                        src_v.at[pl.ds(t8, RPT - t8)])
        pltpu.sync_copy(pd_hbm.at[pl.ds(0, RPT - t8)],
                        dst_v.at[pl.ds(t8, RPT - t8)])
        for j in range(tail, t8):
            for k in range(LANE // 16):
                sl = pl.ds(k * 16, 16)
                src_v[j, sl] = jnp.zeros((16,), jnp.int32)
                dst_v[j, sl] = jnp.full((16,), N + 112 + (j - tail) * 8 + k,
                                        jnp.int32)

    pltpu.sync_copy(asp_hbm, asp_v)
    pltpu.sync_copy(adp_hbm, adp_v)
    pltpu.sync_copy(vp_hbm, vp_v)

    zero16 = jnp.zeros((16,), jnp.float32)

    def zb(i, carry):
        z_v[pl.ds(i * 16, 16)] = zero16
        return carry

    lax.fori_loop(0, SLC // 16, zb, 0)
    pltpu.sync_copy(z_v, acc_s.at[pl.ds(sid * SLC, SLC)])
    pltpu.sync_copy(z_v, acc_n.at[pl.ds(sid * SLC, SLC)])
    plsc.subcore_barrier()

    def row(j, carry):
        for k in range(LANE // 16):
            sl = pl.ds(k * 16, 16)
            si = src_v[j, sl]
            di = dst_v[j, sl]
            a_s = plsc.load_gather(asp_v, [si])
            a_d = plsc.load_gather(adp_v, [di])
            vv = plsc.load_gather(vp_v, [si])
            z = a_s + a_d
            p = jnp.exp(jnp.maximum(z, z * 0.2))
            p_v[j, sl] = p
            q_v[j, sl] = p * vv
        return carry

    lax.fori_loop(0, RPT, row, 0)

    def srow(j, carry):
        pltpu.sync_copy(p_v.at[j], acc_s.at[dst_v.at[j]], add=True)
        pltpu.sync_copy(q_v.at[j], acc_n.at[dst_v.at[j]], add=True)
        return carry

    lax.fori_loop(0, RPT, srow, 0)
    plsc.subcore_barrier()

    pltpu.sync_copy(acc_s.at[pl.ds(sid * SLC, SLC)],
                    s_out.at[c, pl.ds(sid * SLC, SLC)])
    pltpu.sync_copy(acc_n.at[pl.ds(sid * SLC, SLC)],
                    n_out.at[c, pl.ds(sid * SLC, SLC)])


_edge_call = functools.partial(
    pl.kernel,
    out_type=(jax.ShapeDtypeStruct((2, NPAD), jnp.float32),
              jax.ShapeDtypeStruct((2, NPAD), jnp.float32)),
    mesh=plsc.VectorSubcoreMesh(core_axis_name="c", subcore_axis_name="s"),
    compiler_params=pltpu.CompilerParams(needs_layout_passes=False),
    scratch_types=[
        pltpu.VMEM((NPAD,), jnp.float32),
        pltpu.VMEM((NPAD,), jnp.float32),
        pltpu.VMEM((NPAD,), jnp.float32),
        pltpu.VMEM((RPT, LANE), jnp.int32),
        pltpu.VMEM((RPT, LANE), jnp.int32),
        pltpu.VMEM((RPT, LANE), jnp.float32),
        pltpu.VMEM((RPT, LANE), jnp.float32),
        pltpu.VMEM((SLC,), jnp.float32),
        pltpu.VMEM_SHARED((NPAD,), jnp.float32),
        pltpu.VMEM_SHARED((NPAD,), jnp.float32),
    ],
)


def _final_body(s_ref, n_ref, asp_ref, adp_ref, vp_ref,
                b_ref, bias_ref, clsw_ref, clsb_ref, o_ref):
    z = asp_ref[...] + adp_ref[...]
    sp = jnp.exp(jnp.maximum(z, z * 0.2))
    stot = s_ref[0:1, :] + s_ref[1:2, :] + sp
    ntot = n_ref[0:1, :] + n_ref[1:2, :] + sp * vp_ref[...]
    c0 = jnp.sum(bias_ref[...] * clsw_ref[...])
    nv = ntot / (stot + 1e-16) + c0
    batch = b_ref[...]
    nv = jnp.where(batch < NG, nv, 0.0)
    gids = lax.broadcasted_iota(jnp.int32, (NG, NPAD), 0)
    eq = batch == gids
    sums = jnp.sum(jnp.where(eq, nv, 0.0), axis=1)
    cnt = jnp.sum(eq.astype(jnp.float32), axis=1)
    logits = sums / jnp.maximum(cnt, 1.0) + clsb_ref[0, 0]
    o_ref[...] = logits[None, :]


def kernel(x, edge_index, batch, W, a_src, a_dst, bias, cls_W, cls_b):
    f32 = jnp.float32
    P = pl.pallas_call(
        _proj_body,
        out_shape=jax.ShapeDtypeStruct((8, N), f32),
    )(x.astype(f32), W.astype(f32), a_src.reshape(1, D).astype(f32),
      a_dst.reshape(1, D).astype(f32), cls_W.reshape(1, D).astype(f32))
    asp = jnp.pad(P[0], (0, NPAD - N))
    adp = jnp.pad(P[1], (0, NPAD - N))
    vp = jnp.pad(P[2], (0, NPAD - N))

    e3 = jnp.pad(edge_index.astype(jnp.int32).reshape(2, E // LANE, LANE),
                 ((0, 0), (0, 4), (0, 0)))
    pads = jnp.asarray(np.zeros((64, LANE), np.int32))
    padd = jnp.asarray(
        (N + np.arange(64 * LANE, dtype=np.int32) % 112).reshape(64, LANE))

    s_part, n_part = _edge_call(_edge_body)(e3, pads, padd, asp, adp, vp)

    b32 = batch.astype(jnp.int32)
    bpad = jnp.asarray(np.full((NPAD - N,), 1 << 20, np.int32))
    b2 = jnp.concatenate([b32, bpad]).reshape(1, NPAD)
    logits = pl.pallas_call(
        _final_body,
        out_shape=jax.ShapeDtypeStruct((1, NG), f32),
    )(s_part, n_part,
      asp.reshape(1, NPAD), adp.reshape(1, NPAD),
      vp.reshape(1, NPAD), b2,
      bias.reshape(1, D), cls_W.reshape(1, D).astype(f32),
      cls_b.reshape(1, 1).astype(f32))
    return logits.reshape(NG, 1)
